# baseline (speedup 1.0000x reference)
import functools

import jax
import jax.numpy as jnp
from jax import lax
from jax.experimental import pallas as pl
from jax.experimental.pallas import tpu as pltpu
from jax.experimental.pallas import tpu_sc as plsc


def _sc_gather_rows(pred_t, y_idx, B):
    info = plsc.get_sparse_core_info()
    NC, NS, L = info.num_cores, info.num_subcores, info.num_lanes
    NW = NC * NS
    b_per_w = B // NW
    mesh = plsc.VectorSubcoreMesh(core_axis_name="c", subcore_axis_name="s")

    @functools.partial(
        pl.kernel,
        mesh=mesh,
        out_type=jax.ShapeDtypeStruct((B,), jnp.float32),
        scratch_types=[
            pltpu.VMEM((b_per_w,), jnp.int32),
            pltpu.VMEM((b_per_w, 128), jnp.float32),
            pltpu.VMEM((b_per_w,), jnp.float32),
            pltpu.SemaphoreType.DMA,
        ],
    )
    def k(pred_hbm, yidx_hbm, out_hbm, yidx_v, rows_v, gath_v, sem):
        wid = lax.axis_index("s") * NC + lax.axis_index("c")
        base = wid * b_per_w
        col0 = pl.multiple_of((wid // 4) * 128, 128)
        off = (wid % 4) * b_per_w
        pltpu.sync_copy(yidx_hbm.at[pl.ds(base, b_per_w)], yidx_v)
        pltpu.async_copy(pred_hbm.at[yidx_v, pl.ds(col0, 128)], rows_v,
                         sem).wait()
        for g in range(b_per_w // L):
            lanes = lax.iota(jnp.int32, L)

            def body(j, acc):
                return jnp.where(lanes == j,
                                 rows_v[g * L + j, pl.ds(off + g * L, L)],
                                 acc)

            gath_v[pl.ds(g * L, L)] = lax.fori_loop(
                0, L, body, jnp.zeros((L,), jnp.float32))
        pltpu.sync_copy(gath_v, out_hbm.at[pl.ds(base, b_per_w)])

    return k(pred_t, y_idx)


def _tc_body(T, tail_ref, ocr_ref, y_ref, inocr_ref, gath_ref, out_ref):
    match = ocr_ref[...] == y_ref[...][None, :]
    copy_sum = jnp.sum(jnp.where(match, tail_ref[:T, :], 0.0), axis=0)
    vocab = jnp.where(inocr_ref[...], 0.0, gath_ref[...])
    final = copy_sum + vocab
    out_ref[0, 0] = -jnp.sum(jnp.log(final + 1e-45)) / final.shape[0]


def kernel(predictions, y, y_idx, in_ocr, ocr_words):
    B, TOT = predictions.shape
    T = ocr_words.shape[1]
    V = TOT - T

    pred_t = predictions.T
    gathered = _sc_gather_rows(pred_t, y_idx.astype(jnp.int32), B)

    TAIL_BLK = 80
    assert V % TAIL_BLK == 0
    out = pl.pallas_call(
        functools.partial(_tc_body, T),
        grid=(1,),
        in_specs=[
            pl.BlockSpec((TAIL_BLK, B), lambda i: (V // TAIL_BLK, 0)),
            pl.BlockSpec((T, B), lambda i: (0, 0)),
            pl.BlockSpec((B,), lambda i: (0,)),
            pl.BlockSpec((B,), lambda i: (0,)),
            pl.BlockSpec((B,), lambda i: (0,)),
        ],
        out_specs=pl.BlockSpec((1, 1), lambda i: (0, 0),
                               memory_space=pltpu.SMEM),
        out_shape=jax.ShapeDtypeStruct((1, 1), jnp.float32),
    )(
        pred_t,
        ocr_words.astype(jnp.int32).T,
        y.astype(jnp.int32),
        in_ocr,
        gathered,
    )
    return out[0, 0]


# speedup vs baseline: 1.0164x; 1.0164x over previous
import functools

import jax
import jax.numpy as jnp
from jax import lax
from jax.experimental import pallas as pl
from jax.experimental.pallas import tpu as pltpu
from jax.experimental.pallas import tpu_sc as plsc


def _sc_gather_rows(pred_t, y_idx, B):
    info = plsc.get_sparse_core_info()
    NC, NS, L = info.num_cores, info.num_subcores, info.num_lanes
    NW = NC * NS
    b_per_w = B // NW
    mesh = plsc.VectorSubcoreMesh(core_axis_name="c", subcore_axis_name="s")

    @functools.partial(
        pl.kernel,
        mesh=mesh,
        out_type=jax.ShapeDtypeStruct((B,), jnp.float32),
        scratch_types=[
            pltpu.VMEM((b_per_w,), jnp.int32),
            pltpu.VMEM((b_per_w, 128), jnp.float32),
            pltpu.VMEM((b_per_w,), jnp.float32),
            pltpu.SemaphoreType.DMA,
        ],
    )
    def k(pred_hbm, yidx_hbm, out_hbm, yidx_v, rows_v, gath_v, sem):
        wid = lax.axis_index("s") * NC + lax.axis_index("c")
        base = wid * b_per_w
        col0 = pl.multiple_of((wid // 4) * 128, 128)
        off = (wid % 4) * b_per_w
        pltpu.sync_copy(yidx_hbm.at[pl.ds(base, b_per_w)], yidx_v)
        pltpu.async_copy(pred_hbm.at[yidx_v, pl.ds(col0, 128)], rows_v,
                         sem).wait()
        for g in range(b_per_w // L):
            lanes = lax.iota(jnp.int32, L)
            acc = jnp.zeros((L,), jnp.float32)
            for j in range(L):
                acc = jnp.where(lanes == j,
                                rows_v[g * L + j, pl.ds(off + g * L, L)], acc)
            gath_v[pl.ds(g * L, L)] = acc
        pltpu.sync_copy(gath_v, out_hbm.at[pl.ds(base, b_per_w)])

    return k(pred_t, y_idx)


def _tail_body(T, tail_ref, ocr_ref, y_ref, out_ref):
    match = ocr_ref[...] == y_ref[...][None, :]
    out_ref[...] = jnp.sum(jnp.where(match, tail_ref[:T, :], 0.0), axis=0)


def _combine_body(csum_ref, inocr_ref, gath_ref, out_ref):
    final = csum_ref[...] + jnp.where(inocr_ref[...], 0.0, gath_ref[...])
    out_ref[0, 0] = -jnp.sum(jnp.log(final + 1e-45)) / final.shape[0]


def kernel(predictions, y, y_idx, in_ocr, ocr_words):
    B, TOT = predictions.shape
    T = ocr_words.shape[1]
    V = TOT - T

    pred_t = predictions.T
    gathered = _sc_gather_rows(pred_t, y_idx.astype(jnp.int32), B)

    TAIL_BLK = 80
    assert V % TAIL_BLK == 0
    copy_sum = pl.pallas_call(
        functools.partial(_tail_body, T),
        grid=(1,),
        in_specs=[
            pl.BlockSpec((TAIL_BLK, B), lambda i: (V // TAIL_BLK, 0)),
            pl.BlockSpec((T, B), lambda i: (0, 0)),
            pl.BlockSpec((B,), lambda i: (0,)),
        ],
        out_specs=pl.BlockSpec((B,), lambda i: (0,)),
        out_shape=jax.ShapeDtypeStruct((B,), jnp.float32),
    )(pred_t, ocr_words.astype(jnp.int32).T, y.astype(jnp.int32))

    out = pl.pallas_call(
        _combine_body,
        out_shape=jax.ShapeDtypeStruct((1, 1), jnp.float32),
        out_specs=pl.BlockSpec(memory_space=pltpu.SMEM),
    )(copy_sum, in_ocr, gathered)
    return out[0, 0]


# speedup vs baseline: 1.0675x; 1.0503x over previous
import functools

import jax
import jax.numpy as jnp
from jax import lax
from jax.experimental import pallas as pl
from jax.experimental.pallas import tpu as pltpu
from jax.experimental.pallas import tpu_sc as plsc


def _sc_gather_rows(pred_t, y_idx, B):
    info = plsc.get_sparse_core_info()
    NS, L = info.num_subcores, info.num_lanes
    NW = NS
    b_per_w = B // NW
    mesh = plsc.VectorSubcoreMesh(core_axis_name="c", subcore_axis_name="s",
                                  num_cores=1)

    @functools.partial(
        pl.kernel,
        mesh=mesh,
        out_type=jax.ShapeDtypeStruct((B,), jnp.float32),
        scratch_types=[
            pltpu.VMEM((b_per_w,), jnp.int32),
            pltpu.VMEM((b_per_w, 128), jnp.float32),
            pltpu.VMEM((b_per_w,), jnp.float32),
            pltpu.SemaphoreType.DMA,
        ],
    )
    def k(pred_hbm, yidx_hbm, out_hbm, yidx_v, rows_v, gath_v, sem):
        wid = lax.axis_index("s")
        base = wid * b_per_w
        col0 = pl.multiple_of((wid // 2) * 128, 128)
        off = (wid % 2) * b_per_w
        pltpu.sync_copy(yidx_hbm.at[pl.ds(base, b_per_w)], yidx_v)
        pltpu.async_copy(pred_hbm.at[yidx_v, pl.ds(col0, 128)], rows_v,
                         sem).wait()
        for g in range(b_per_w // L):
            lanes = lax.iota(jnp.int32, L)
            acc = jnp.zeros((L,), jnp.float32)
            for j in range(L):
                acc = jnp.where(lanes == j,
                                rows_v[g * L + j, pl.ds(off + g * L, L)], acc)
            gath_v[pl.ds(g * L, L)] = acc
        pltpu.sync_copy(gath_v, out_hbm.at[pl.ds(base, b_per_w)])

    return k(pred_t, y_idx)


def _tail_body(T, tail_ref, ocr_ref, y_ref, out_ref):
    match = ocr_ref[...] == y_ref[...][None, :]
    out_ref[...] = jnp.sum(jnp.where(match, tail_ref[:T, :], 0.0), axis=0)


def _combine_body(csum_ref, inocr_ref, gath_ref, out_ref):
    final = csum_ref[...] + jnp.where(inocr_ref[...], 0.0, gath_ref[...])
    out_ref[0, 0] = -jnp.sum(jnp.log(final + 1e-45)) / final.shape[0]


def kernel(predictions, y, y_idx, in_ocr, ocr_words):
    B, TOT = predictions.shape
    T = ocr_words.shape[1]
    V = TOT - T

    pred_t = predictions.T
    gathered = _sc_gather_rows(pred_t, y_idx.astype(jnp.int32), B)

    TAIL_BLK = 80
    assert V % TAIL_BLK == 0
    copy_sum = pl.pallas_call(
        functools.partial(_tail_body, T),
        grid=(1,),
        in_specs=[
            pl.BlockSpec((TAIL_BLK, B), lambda i: (V // TAIL_BLK, 0)),
            pl.BlockSpec((T, B), lambda i: (0, 0)),
            pl.BlockSpec((B,), lambda i: (0,)),
        ],
        out_specs=pl.BlockSpec((B,), lambda i: (0,)),
        out_shape=jax.ShapeDtypeStruct((B,), jnp.float32),
    )(pred_t, ocr_words.astype(jnp.int32).T, y.astype(jnp.int32))

    out = pl.pallas_call(
        _combine_body,
        out_shape=jax.ShapeDtypeStruct((1, 1), jnp.float32),
        out_specs=pl.BlockSpec(memory_space=pltpu.SMEM),
    )(copy_sum, in_ocr, gathered)
    return out[0, 0]
